# Initial kernel scaffold; baseline (speedup 1.0000x reference)
import jax
import jax.numpy as jnp
from jax.experimental import pallas as pl


def kernel(energy_readout, atomic_numbers, atomic_subsystem_indices, self_energies_tensor):
    raise NotImplementedError("write your pallas kernel here")


# speedup vs baseline: 14.8988x; 14.8988x over previous
import functools

import jax
import jax.numpy as jnp
from jax import lax
from jax.experimental import pallas as pl
from jax.experimental.pallas import tpu as pltpu
from jax.experimental.pallas import tpu_sc as plsc

NC = 2
NS = 16
NW = NC * NS
LANES = 16
BLKM = 2048


def _phase1(num_blocks, chunk, atom_blk, mpad):
    mesh = plsc.VectorSubcoreMesh(core_axis_name="c", subcore_axis_name="s")
    inner_iters = atom_blk // (5 * LANES)

    @functools.partial(
        pl.kernel,
        mesh=mesh,
        compiler_params=pltpu.CompilerParams(needs_layout_passes=False),
        out_type=jax.ShapeDtypeStruct((mpad // BLKM, NW, BLKM), jnp.float32),
        scratch_types=[
            pltpu.VMEM((LANES,), jnp.float32),
            pltpu.VMEM((atom_blk,), jnp.int32),
            pltpu.VMEM((atom_blk,), jnp.int32),
            pltpu.VMEM((mpad,), jnp.float32),
        ],
    )
    def k(z_hbm, ids_hbm, table_hbm, part_hbm, table_v, z_buf, ids_buf, acc):
        wid = lax.axis_index("s") * NC + lax.axis_index("c")
        pltpu.sync_copy(table_hbm, table_v)

        zeros = jnp.zeros((LANES,), jnp.float32)

        def zbody(j, c):
            for u in range(8):
                acc[pl.ds(j * 8 * LANES + u * LANES, LANES)] = zeros
            return c

        lax.fori_loop(0, mpad // (8 * LANES), zbody, 0)

        base = pl.multiple_of(wid * chunk, 8)

        def blk(b, c):
            off = pl.multiple_of(base + b * atom_blk, 8)
            pltpu.sync_copy(z_hbm.at[pl.ds(off, atom_blk)], z_buf)
            pltpu.sync_copy(ids_hbm.at[pl.ds(off, atom_blk)], ids_buf)

            def inner(i, ci):
                for u in range(5):
                    o = i * 5 * LANES + u * LANES
                    z16 = z_buf[pl.ds(o, LANES)]
                    s16 = ids_buf[pl.ds(o, LANES)]
                    v = plsc.load_gather(table_v, [z16])
                    plsc.addupdate_scatter(acc, [s16], v)
                return ci

            lax.fori_loop(0, inner_iters, inner, 0)
            return c

        lax.fori_loop(0, num_blocks, blk, 0)

        def wout(j, c):
            pltpu.sync_copy(acc.at[pl.ds(j * BLKM, BLKM)], part_hbm.at[j, wid])
            return c

        lax.fori_loop(0, mpad // BLKM, wout, 0)

    return k


def _p2body(part_ref, ro_ref, out_ref):
    s = jnp.sum(part_ref[0], axis=0, keepdims=True)
    out_ref[0] = ro_ref[0] + s


def kernel(energy_readout, atomic_numbers, atomic_subsystem_indices,
           self_energies_tensor):
    n = atomic_numbers.shape[0]
    m = energy_readout.shape[0]
    chunk = n // NW
    atom_blk = 4000
    num_blocks = chunk // atom_blk
    mpad = ((m + BLKM - 1) // BLKM) * BLKM
    mb = mpad // BLKM

    table16 = jnp.pad(self_energies_tensor,
                      (0, LANES - self_energies_tensor.shape[0]))

    partials = _phase1(num_blocks, chunk, atom_blk, mpad)(
        atomic_numbers, atomic_subsystem_indices, table16)

    ro = jnp.pad(energy_readout, (0, mpad - m)).reshape(mb, 1, BLKM)

    out = pl.pallas_call(
        _p2body,
        grid=(mb,),
        in_specs=[
            pl.BlockSpec((1, NW, BLKM), lambda i: (i, 0, 0)),
            pl.BlockSpec((1, 1, BLKM), lambda i: (i, 0, 0)),
        ],
        out_specs=pl.BlockSpec((1, 1, BLKM), lambda i: (i, 0, 0)),
        out_shape=jax.ShapeDtypeStruct((mb, 1, BLKM), jnp.float32),
    )(partials, ro)

    return out.reshape(mpad)[:m]


# speedup vs baseline: 17.7299x; 1.1900x over previous
import functools

import jax
import jax.numpy as jnp
from jax import lax
from jax.experimental import pallas as pl
from jax.experimental.pallas import tpu as pltpu
from jax.experimental.pallas import tpu_sc as plsc

NC = 2
NS = 16
NW = NC * NS
LANES = 16
UNROLL = 5
BLKM = 2048


def _phase1(num_blocks, chunk, atom_blk, mpad):
    mesh = plsc.VectorSubcoreMesh(core_axis_name="c", subcore_axis_name="s")
    inner_iters = atom_blk // (UNROLL * LANES)
    assert num_blocks % 2 == 1 and num_blocks >= 3

    @functools.partial(
        pl.kernel,
        mesh=mesh,
        compiler_params=pltpu.CompilerParams(needs_layout_passes=False),
        out_type=jax.ShapeDtypeStruct((NW, mpad), jnp.float32),
        scratch_types=[
            pltpu.VMEM((LANES,), jnp.float32),
            pltpu.VMEM((atom_blk,), jnp.int32),
            pltpu.VMEM((atom_blk,), jnp.int32),
            pltpu.VMEM((atom_blk,), jnp.int32),
            pltpu.VMEM((atom_blk,), jnp.int32),
            pltpu.VMEM((mpad,), jnp.float32),
            pltpu.SemaphoreType.DMA,
            pltpu.SemaphoreType.DMA,
            pltpu.SemaphoreType.DMA,
            pltpu.SemaphoreType.DMA,
        ],
    )
    def k(z_hbm, ids_hbm, table_hbm, part_hbm,
          table_v, z0, i0, z1, i1, acc, sz0, si0, sz1, si1):
        wid = lax.axis_index("s") * NC + lax.axis_index("c")
        pltpu.sync_copy(table_hbm, table_v)

        zeros = jnp.zeros((LANES,), jnp.float32)

        def zbody(j, c):
            for u in range(8):
                acc[pl.ds(j * 8 * LANES + u * LANES, LANES)] = zeros
            return c

        lax.fori_loop(0, mpad // (8 * LANES), zbody, 0)

        base = pl.multiple_of(wid * chunk, 8)

        def start(b, zbuf, ibuf, zsem, isem):
            off = pl.multiple_of(base + b * atom_blk, 8)
            pltpu.async_copy(z_hbm.at[pl.ds(off, atom_blk)], zbuf, zsem)
            pltpu.async_copy(ids_hbm.at[pl.ds(off, atom_blk)], ibuf, isem)

        def wait(zbuf, ibuf, zsem, isem):
            pltpu.make_async_copy(
                z_hbm.at[pl.ds(0, atom_blk)], zbuf, zsem).wait()
            pltpu.make_async_copy(
                ids_hbm.at[pl.ds(0, atom_blk)], ibuf, isem).wait()

        def compute(zbuf, ibuf):
            def inner(i, ci):
                for u in range(UNROLL):
                    o = i * UNROLL * LANES + u * LANES
                    z16 = zbuf[pl.ds(o, LANES)]
                    s16 = ibuf[pl.ds(o, LANES)]
                    v = plsc.load_gather(table_v, [z16])
                    plsc.addupdate_scatter(acc, [s16], v)
                return ci

            lax.fori_loop(0, inner_iters, inner, 0)

        start(0, z0, i0, sz0, si0)

        def super_step(j, c):
            b = j * 2
            start(b + 1, z1, i1, sz1, si1)
            wait(z0, i0, sz0, si0)
            compute(z0, i0)
            start(b + 2, z0, i0, sz0, si0)
            wait(z1, i1, sz1, si1)
            compute(z1, i1)
            return c

        lax.fori_loop(0, (num_blocks - 1) // 2, super_step, 0)
        wait(z0, i0, sz0, si0)
        compute(z0, i0)

        pltpu.sync_copy(acc, part_hbm.at[wid])

    return k


def _p2body(part_ref, ro_ref, out_ref):
    out_ref[...] = ro_ref[...] + jnp.sum(part_ref[...], axis=0, keepdims=True)


def kernel(energy_readout, atomic_numbers, atomic_subsystem_indices,
           self_energies_tensor):
    n = atomic_numbers.shape[0]
    m = energy_readout.shape[0]
    chunk = n // NW
    atom_blk = 4000
    num_blocks = chunk // atom_blk
    mpad = ((m + BLKM - 1) // BLKM) * BLKM
    mb = mpad // BLKM

    table16 = jnp.pad(self_energies_tensor,
                      (0, LANES - self_energies_tensor.shape[0]))

    partials = _phase1(num_blocks, chunk, atom_blk, mpad)(
        atomic_numbers, atomic_subsystem_indices, table16)

    ro = jnp.pad(energy_readout, (0, mpad - m)).reshape(1, mpad)

    out = pl.pallas_call(
        _p2body,
        grid=(mb,),
        in_specs=[
            pl.BlockSpec((NW, BLKM), lambda i: (0, i)),
            pl.BlockSpec((1, BLKM), lambda i: (0, i)),
        ],
        out_specs=pl.BlockSpec((1, BLKM), lambda i: (0, i)),
        out_shape=jax.ShapeDtypeStruct((1, mpad), jnp.float32),
    )(partials, ro)

    return out.reshape(mpad)[:m]
